# Initial kernel scaffold; baseline (speedup 1.0000x reference)
import jax
import jax.numpy as jnp
from jax.experimental import pallas as pl


def kernel(tenIn, tenFlow, tenMetric):
    raise NotImplementedError("write your pallas kernel here")


# speedup vs baseline: 2.6693x; 2.6693x over previous
import functools

import jax
import jax.numpy as jnp
from jax import lax
from jax.experimental import pallas as pl
from jax.experimental.pallas import tpu as pltpu
from jax.experimental.pallas import tpu_sc as plsc

N, C, H, W = 2, 96, 384, 384
CP = 128
NHW = N * H * W
SR = 16
G = 16
HALO = 8
NSLAB_IMG = H // SR
NSLAB = N * NSLAB_IMG
DUMP = G + SR * W + 32
BUF_ROWS = 7168
GSZ = 128
HSZ = 64
SRC_ROWS = SR + 2 * HALO
PER_TILE = SRC_ROWS * W // 16
NB = PER_TILE // GSZ
RB = 8


def _prep_body(flow_ref, met_ref, wgt_ref, yx_ref):
    rc = pl.program_id(1)
    fx = flow_ref[0, 0]
    fy = flow_ref[0, 1]
    m = met_ref[0, 0]
    xg = lax.broadcasted_iota(jnp.int32, (RB, W), 1).astype(jnp.float32)
    yg = (lax.broadcasted_iota(jnp.int32, (RB, W), 0)
          .astype(jnp.float32) + (rc * RB).astype(jnp.float32))
    fltx = xg + fx
    flty = yg + fy
    x0f = jnp.floor(fltx)
    y0f = jnp.floor(flty)
    x0 = x0f.astype(jnp.int32)
    y0 = y0f.astype(jnp.int32)
    ax = fltx - x0f
    ay = flty - y0f
    e = jnp.exp(m)
    vx0 = (x0 >= 0) & (x0 < W)
    vx1 = (x0 >= -1) & (x0 < W - 1)
    bx = 1.0 - ax
    by = 1.0 - ay
    z = jnp.zeros_like(ax)
    wNW = jnp.where(vx0, bx * by * e, z)
    wNE = jnp.where(vx1, ax * by * e, z)
    wSW = jnp.where(vx0, bx * ay * e, z)
    wSE = jnp.where(vx1, ax * ay * e, z)
    xc = jnp.clip(x0, -8, W + 7)
    P = RB * W
    wgt_ref[...] = jnp.concatenate(
        [wNW.reshape(1, P), wNE.reshape(1, P), wSW.reshape(1, P),
         wSE.reshape(1, P), jnp.zeros((4, P), jnp.float32)], axis=0)
    yx_ref[...] = jnp.concatenate(
        [y0.reshape(1, P), xc.reshape(1, P), jnp.zeros((6, P), jnp.int32)],
        axis=0)


def _prep(tenFlow, tenMetric):
    return pl.pallas_call(
        _prep_body,
        grid=(N, H // RB),
        in_specs=[pl.BlockSpec((1, 2, RB, W), lambda n, r: (n, 0, r, 0)),
                  pl.BlockSpec((1, 1, RB, W), lambda n, r: (n, 0, r, 0))],
        out_specs=[pl.BlockSpec((8, RB * W), lambda n, r: (0, n * (H // RB) + r)),
                   pl.BlockSpec((8, RB * W), lambda n, r: (0, n * (H // RB) + r))],
        out_shape=[jax.ShapeDtypeStruct((8, NHW), jnp.float32),
                   jax.ShapeDtypeStruct((8, NHW), jnp.int32)],
    )(tenFlow, tenMetric)


def _sc_splat_body(in_hbm, wgt_hbm, yx_hbm, out_hbm,
                   vbuf, wbuf, ybuf, sidx, stage, zbuf, acc):
    cid = lax.axis_index("c")
    tid = lax.axis_index("s")
    zvec = jnp.zeros((16,), jnp.float32)

    @pl.loop(0, 56)
    def _zb(i):
        for t in range(CP // 16):
            zbuf[i, pl.ds(16 * t, 16)] = zvec

    @pl.loop(0, HSZ)
    def _zs(i):
        for j in range(4):
            stage[j, i, pl.ds(CP - 16, 16)] = zvec

    @pl.loop(0, NSLAB // 2)
    def _slab(si):
        s = si * 2 + cid
        n = s // NSLAB_IMG
        r0 = (s % NSLAB_IMG) * SR
        for zi in range(BUF_ROWS // 16 // 56):
            pltpu.sync_copy(
                zbuf, acc.at[pl.ds(tid * (BUF_ROWS // 16) + zi * 56, 56)])
        plsc.subcore_barrier()
        lo = lax.clamp(0, r0 - HALO, H - SRC_ROWS)
        start = n * (H * W) + lo * W + tid * PER_TILE

        @pl.loop(0, NB)
        def _batch(b):
            p = start + b * GSZ
            pltpu.sync_copy(in_hbm.at[pl.ds(p, GSZ)], vbuf)
            pltpu.sync_copy(wgt_hbm.at[pl.ds(0, 8), pl.ds(p, GSZ)], wbuf)
            pltpu.sync_copy(yx_hbm.at[pl.ds(0, 8), pl.ds(p, GSZ)], ybuf)
            e0 = jnp.where(lax.iota(jnp.int32, 16) == 0,
                           jnp.float32(1.0), jnp.float32(0.0))
            dump = jnp.full((16,), DUMP, jnp.int32)
            for h in range(GSZ // HSZ):
                for k in range(HSZ // 16):
                    kk = h * (HSZ // 16) + k
                    y0v = ybuf[0, pl.ds(16 * kk, 16)]
                    xcv = ybuf[1, pl.ds(16 * kk, 16)]
                    la = (y0v - r0) * W + xcv + G
                    okA = (y0v >= r0) & (y0v < r0 + SR)
                    okB = (y0v >= r0 - 1) & (y0v < r0 + SR - 1)
                    iA = jnp.where(okA, la, dump)
                    iB = jnp.where(okB, la + W, dump)
                    sidx[0, pl.ds(16 * k, 16)] = iA
                    sidx[1, pl.ds(16 * k, 16)] = iA + 1
                    sidx[2, pl.ds(16 * k, 16)] = iB
                    sidx[3, pl.ds(16 * k, 16)] = iB + 1

                @pl.loop(0, HSZ // 16)
                def _grp(g):
                    w0v = wbuf[0, pl.ds(h * HSZ + g * 16, 16)]
                    w1v = wbuf[1, pl.ds(h * HSZ + g * 16, 16)]
                    w2v = wbuf[2, pl.ds(h * HSZ + g * 16, 16)]
                    w3v = wbuf[3, pl.ds(h * HSZ + g * 16, 16)]
                    for k in range(16):
                        qv = h * HSZ + g * 16 + k
                        qs = g * 16 + k
                        w0 = w0v[k]
                        w1 = w1v[k]
                        w2 = w2v[k]
                        w3 = w3v[k]
                        for t in range(C // 16):
                            vv = vbuf[qv, pl.ds(16 * t, 16)]
                            stage[0, qs, pl.ds(16 * t, 16)] = vv * w0
                            stage[1, qs, pl.ds(16 * t, 16)] = vv * w1
                            stage[2, qs, pl.ds(16 * t, 16)] = vv * w2
                            stage[3, qs, pl.ds(16 * t, 16)] = vv * w3
                        stage[0, qs, pl.ds(C, 16)] = e0 * w0
                        stage[1, qs, pl.ds(C, 16)] = e0 * w1
                        stage[2, qs, pl.ds(C, 16)] = e0 * w2
                        stage[3, qs, pl.ds(C, 16)] = e0 * w3

                for j in range(4):
                    pltpu.sync_copy(stage.at[j], acc.at[sidx.at[j]], add=True)

        plsc.subcore_barrier()
        out_base = n * (H * W) + r0 * W + tid * (SR * W // 16)
        for fi in range(SR * W // 16 // GSZ):
            pltpu.sync_copy(
                acc.at[pl.ds(G + tid * (SR * W // 16) + fi * GSZ, GSZ)],
                out_hbm.at[pl.ds(out_base + fi * GSZ, GSZ)])
        plsc.subcore_barrier()


def _sc_splat(in_nhwc, wgt, yx):
    mesh = plsc.VectorSubcoreMesh(core_axis_name="c", subcore_axis_name="s")
    k = functools.partial(
        pl.kernel, mesh=mesh,
        out_type=jax.ShapeDtypeStruct((NHW, CP), jnp.float32),
        scratch_types=[
            pltpu.VMEM((GSZ, C), jnp.float32),
            pltpu.VMEM((8, GSZ), jnp.float32),
            pltpu.VMEM((8, GSZ), jnp.int32),
            pltpu.VMEM((4, HSZ), jnp.int32),
            pltpu.VMEM((4, HSZ, CP), jnp.float32),
            pltpu.VMEM((56, CP), jnp.float32),
            pltpu.VMEM_SHARED((BUF_ROWS, CP), jnp.float32),
        ],
    )(_sc_splat_body)
    return k(in_nhwc, wgt, yx)


def _post_body(acc_ref, out_ref):
    a = acc_ref[...]
    norm = a[:, C:C + 1] + 1e-7
    out_ref[...] = a[:, :C] / norm


def _post(outp):
    RB2 = 1024
    return pl.pallas_call(
        _post_body,
        grid=(NHW // RB2,),
        in_specs=[pl.BlockSpec((RB2, CP), lambda i: (i, 0))],
        out_specs=pl.BlockSpec((RB2, C), lambda i: (i, 0)),
        out_shape=jax.ShapeDtypeStruct((NHW, C), jnp.float32),
    )(outp)


def kernel(tenIn, tenFlow, tenMetric):
    in_nhwc = tenIn.transpose(0, 2, 3, 1).reshape(NHW, C)
    wgt, yx = _prep(tenFlow, tenMetric)
    outp = _sc_splat(in_nhwc, wgt, yx)
    res = _post(outp)
    return res.reshape(N, H, W, C).transpose(0, 3, 1, 2)


# speedup vs baseline: 2.8106x; 1.0529x over previous
import functools

import jax
import jax.numpy as jnp
from jax import lax
from jax.experimental import pallas as pl
from jax.experimental.pallas import tpu as pltpu
from jax.experimental.pallas import tpu_sc as plsc

N, C, H, W = 2, 96, 384, 384
CP = 128
NHW = N * H * W
SR = 16
G = 16
HALO = 8
NSLAB_IMG = H // SR
NSLAB = N * NSLAB_IMG
DUMP = G + SR * W + 32
BUF_ROWS = 6272
GSZ = 128
HSZ = 32
SRC_ROWS = SR + 2 * HALO
PER_TILE = SRC_ROWS * W // 16
NB = PER_TILE // GSZ
RB = 8


def _prep_body(flow_ref, met_ref, wgt_ref, yx_ref):
    rc = pl.program_id(1)
    fx = flow_ref[0, 0]
    fy = flow_ref[0, 1]
    m = met_ref[0, 0]
    xg = lax.broadcasted_iota(jnp.int32, (RB, W), 1).astype(jnp.float32)
    yg = (lax.broadcasted_iota(jnp.int32, (RB, W), 0)
          .astype(jnp.float32) + (rc * RB).astype(jnp.float32))
    fltx = xg + fx
    flty = yg + fy
    x0f = jnp.floor(fltx)
    y0f = jnp.floor(flty)
    x0 = x0f.astype(jnp.int32)
    y0 = y0f.astype(jnp.int32)
    ax = fltx - x0f
    ay = flty - y0f
    e = jnp.exp(m)
    vx0 = (x0 >= 0) & (x0 < W)
    vx1 = (x0 >= -1) & (x0 < W - 1)
    bx = 1.0 - ax
    by = 1.0 - ay
    z = jnp.zeros_like(ax)
    wNW = jnp.where(vx0, bx * by * e, z)
    wNE = jnp.where(vx1, ax * by * e, z)
    wSW = jnp.where(vx0, bx * ay * e, z)
    wSE = jnp.where(vx1, ax * ay * e, z)
    xc = jnp.clip(x0, -8, W + 7)
    P = RB * W
    wgt_ref[...] = jnp.concatenate(
        [wNW.reshape(1, P), wNE.reshape(1, P), wSW.reshape(1, P),
         wSE.reshape(1, P), jnp.zeros((4, P), jnp.float32)], axis=0)
    yx_ref[...] = jnp.concatenate(
        [y0.reshape(1, P), xc.reshape(1, P), jnp.zeros((6, P), jnp.int32)],
        axis=0)


def _prep(tenFlow, tenMetric):
    return pl.pallas_call(
        _prep_body,
        grid=(N, H // RB),
        in_specs=[pl.BlockSpec((1, 2, RB, W), lambda n, r: (n, 0, r, 0)),
                  pl.BlockSpec((1, 1, RB, W), lambda n, r: (n, 0, r, 0))],
        out_specs=[pl.BlockSpec((8, RB * W), lambda n, r: (0, n * (H // RB) + r)),
                   pl.BlockSpec((8, RB * W), lambda n, r: (0, n * (H // RB) + r))],
        out_shape=[jax.ShapeDtypeStruct((8, NHW), jnp.float32),
                   jax.ShapeDtypeStruct((8, NHW), jnp.int32)],
    )(tenFlow, tenMetric)


def _sc_splat_body(in_hbm, wgt_hbm, yx_hbm, out_hbm,
                   vbuf, wbuf, ybuf, ix0, ix1, st0, st1, zbuf, acc,
                   sem_in, sem_s0, sem_s1):
    cid = lax.axis_index("c")
    tid = lax.axis_index("s")
    zvec = jnp.zeros((16,), jnp.float32)

    @pl.loop(0, 28)
    def _zb(i):
        for t in range(CP // 16):
            zbuf[i, pl.ds(16 * t, 16)] = zvec

    @pl.loop(0, 4 * HSZ)
    def _zs(i):
        st0[i, pl.ds(CP - 16, 16)] = zvec
        st1[i, pl.ds(CP - 16, 16)] = zvec

    def start_in(pv, voff, wrow):
        pltpu.async_copy(in_hbm.at[pl.ds(pv, GSZ)],
                         vbuf.at[pl.ds(voff, GSZ)], sem_in)
        pltpu.async_copy(wgt_hbm.at[pl.ds(0, 8), pl.ds(pv, GSZ)],
                         wbuf.at[pl.ds(wrow, 8)], sem_in)
        pltpu.async_copy(yx_hbm.at[pl.ds(0, 8), pl.ds(pv, GSZ)],
                         ybuf.at[pl.ds(wrow, 8)], sem_in)

    def wait_in(pv, voff, wrow):
        pltpu.make_async_copy(in_hbm.at[pl.ds(pv, GSZ)],
                              vbuf.at[pl.ds(voff, GSZ)], sem_in).wait()
        pltpu.make_async_copy(wgt_hbm.at[pl.ds(0, 8), pl.ds(pv, GSZ)],
                              wbuf.at[pl.ds(wrow, 8)], sem_in).wait()
        pltpu.make_async_copy(yx_hbm.at[pl.ds(0, 8), pl.ds(pv, GSZ)],
                              ybuf.at[pl.ds(wrow, 8)], sem_in).wait()

    def fill_half(S, IX, h, voff, wrow, r0):
        dump = jnp.full((16,), DUMP, jnp.int32)
        e0 = jnp.where(lax.iota(jnp.int32, 16) == 0,
                       jnp.float32(1.0), jnp.float32(0.0))
        for k in range(HSZ // 16):
            cs = h * HSZ + 16 * k
            y0v = ybuf[wrow, pl.ds(cs, 16)]
            xcv = ybuf[wrow + 1, pl.ds(cs, 16)]
            la = (y0v - r0) * W + xcv + G
            okA = (y0v >= r0) & (y0v < r0 + SR)
            okB = (y0v >= r0 - 1) & (y0v < r0 + SR - 1)
            iA = jnp.where(okA, la, dump)
            iB = jnp.where(okB, la + W, dump)
            IX[0, pl.ds(16 * k, 16)] = iA
            IX[0, pl.ds(HSZ + 16 * k, 16)] = iA + 1
            IX[0, pl.ds(2 * HSZ + 16 * k, 16)] = iB
            IX[0, pl.ds(3 * HSZ + 16 * k, 16)] = iB + 1

        @pl.loop(0, HSZ // 16)
        def _grp(g):
            base = h * HSZ + g * 16
            w0v = wbuf[wrow, pl.ds(base, 16)]
            w1v = wbuf[wrow + 1, pl.ds(base, 16)]
            w2v = wbuf[wrow + 2, pl.ds(base, 16)]
            w3v = wbuf[wrow + 3, pl.ds(base, 16)]
            for k in range(16):
                qv = voff + base + k
                qs = g * 16 + k
                w0 = w0v[k]
                w1 = w1v[k]
                w2 = w2v[k]
                w3 = w3v[k]
                for t in range(C // 16):
                    vv = vbuf[qv, pl.ds(16 * t, 16)]
                    st = pl.ds(16 * t, 16)
                    S[qs, st] = vv * w0
                    S[HSZ + qs, st] = vv * w1
                    S[2 * HSZ + qs, st] = vv * w2
                    S[3 * HSZ + qs, st] = vv * w3
                S[qs, pl.ds(C, 16)] = e0 * w0
                S[HSZ + qs, pl.ds(C, 16)] = e0 * w1
                S[2 * HSZ + qs, pl.ds(C, 16)] = e0 * w2
                S[3 * HSZ + qs, pl.ds(C, 16)] = e0 * w3

    @pl.loop(0, NSLAB // 2)
    def _slab(si):
        s = si * 2 + cid
        n = s // NSLAB_IMG
        r0 = (s % NSLAB_IMG) * SR
        for zi in range(BUF_ROWS // 16 // 28):
            pltpu.sync_copy(
                zbuf, acc.at[pl.ds(tid * (BUF_ROWS // 16) + zi * 28, 28)])
        plsc.subcore_barrier()
        lo = lax.clamp(0, r0 - HALO, H - SRC_ROWS)
        start = n * (H * W) + lo * W + tid * PER_TILE
        start_in(start, 0, 0)

        @pl.loop(0, NB)
        def _batch(b):
            pv = start + b * GSZ
            par = b % 2
            voff = par * GSZ
            wrow = par * 8
            wait_in(pv, voff, wrow)

            @pl.when(b + 1 < NB)
            def _():
                par2 = (b + 1) % 2
                start_in(pv + GSZ, par2 * GSZ, par2 * 8)

            @pl.loop(0, 2)
            def _hp(hp):
                for sp, (S, IX, SM) in enumerate(
                        ((st0, ix0, sem_s0), (st1, ix1, sem_s1))):
                    h = hp * 2 + sp

                    @pl.when((b > 0) | (hp > 0))
                    def _():
                        pltpu.make_async_copy(
                            S, acc.at[IX.at[0]], SM).wait()

                    fill_half(S, IX, h, voff, wrow, r0)
                    pltpu.async_copy(S, acc.at[IX.at[0]], SM, add=True)

        pltpu.make_async_copy(st0, acc.at[ix0.at[0]], sem_s0).wait()
        pltpu.make_async_copy(st1, acc.at[ix1.at[0]], sem_s1).wait()
        plsc.subcore_barrier()
        out_base = n * (H * W) + r0 * W + tid * (SR * W // 16)
        for fi in range(SR * W // 16 // GSZ):
            pltpu.sync_copy(
                acc.at[pl.ds(G + tid * (SR * W // 16) + fi * GSZ, GSZ)],
                out_hbm.at[pl.ds(out_base + fi * GSZ, GSZ)])
        plsc.subcore_barrier()


def _sc_splat(in_nhwc, wgt, yx):
    mesh = plsc.VectorSubcoreMesh(core_axis_name="c", subcore_axis_name="s")
    k = functools.partial(
        pl.kernel, mesh=mesh,
        out_type=jax.ShapeDtypeStruct((NHW, CP), jnp.float32),
        scratch_types=[
            pltpu.VMEM((2 * GSZ, C), jnp.float32),
            pltpu.VMEM((16, GSZ), jnp.float32),
            pltpu.VMEM((16, GSZ), jnp.int32),
            pltpu.VMEM((1, 4 * HSZ), jnp.int32),
            pltpu.VMEM((1, 4 * HSZ), jnp.int32),
            pltpu.VMEM((4 * HSZ, CP), jnp.float32),
            pltpu.VMEM((4 * HSZ, CP), jnp.float32),
            pltpu.VMEM((28, CP), jnp.float32),
            pltpu.VMEM_SHARED((BUF_ROWS, CP), jnp.float32),
            pltpu.SemaphoreType.DMA,
            pltpu.SemaphoreType.DMA,
            pltpu.SemaphoreType.DMA,
        ],    )(_sc_splat_body)
    return k(in_nhwc, wgt, yx)


def _post_body(acc_ref, out_ref):
    a = acc_ref[...]
    norm = a[:, C:C + 1] + 1e-7
    out_ref[...] = a[:, :C] / norm


def _post(outp):
    RB2 = 1024
    return pl.pallas_call(
        _post_body,
        grid=(NHW // RB2,),
        in_specs=[pl.BlockSpec((RB2, CP), lambda i: (i, 0))],
        out_specs=pl.BlockSpec((RB2, C), lambda i: (i, 0)),
        out_shape=jax.ShapeDtypeStruct((NHW, C), jnp.float32),
    )(outp)


def kernel(tenIn, tenFlow, tenMetric):
    in_nhwc = tenIn.transpose(0, 2, 3, 1).reshape(NHW, C)
    wgt, yx = _prep(tenFlow, tenMetric)
    outp = _sc_splat(in_nhwc, wgt, yx)
    res = _post(outp)
    return res.reshape(N, H, W, C).transpose(0, 3, 1, 2)


# speedup vs baseline: 3.4944x; 1.2433x over previous
import functools

import jax
import jax.numpy as jnp
from jax import lax
from jax.experimental import pallas as pl
from jax.experimental.pallas import tpu as pltpu
from jax.experimental.pallas import tpu_sc as plsc

N, C, H, W = 2, 96, 384, 384
CP = 128
NHW = N * H * W
SR = 16
G = 16
HALO = 8
NSLAB_IMG = H // SR
NSLAB = N * NSLAB_IMG
DUMP = G + SR * W + 32
BUF_ROWS = 6272
GSZ = 128
HSZ = 32
SRC_ROWS = SR + 2 * HALO
PER_TILE = SRC_ROWS * W // 16
NB = PER_TILE // GSZ
RB = 8


def _prep_body(flow_ref, met_ref, wgt_ref, yx_ref):
    rc = pl.program_id(1)
    fx = flow_ref[0, 0]
    fy = flow_ref[0, 1]
    m = met_ref[0, 0]
    xg = lax.broadcasted_iota(jnp.int32, (RB, W), 1).astype(jnp.float32)
    yg = (lax.broadcasted_iota(jnp.int32, (RB, W), 0)
          .astype(jnp.float32) + (rc * RB).astype(jnp.float32))
    fltx = xg + fx
    flty = yg + fy
    x0f = jnp.floor(fltx)
    y0f = jnp.floor(flty)
    x0 = x0f.astype(jnp.int32)
    y0 = y0f.astype(jnp.int32)
    ax = fltx - x0f
    ay = flty - y0f
    e = jnp.exp(m)
    vx0 = (x0 >= 0) & (x0 < W)
    vx1 = (x0 >= -1) & (x0 < W - 1)
    bx = 1.0 - ax
    by = 1.0 - ay
    z = jnp.zeros_like(ax)
    wNW = jnp.where(vx0, bx * by * e, z)
    wNE = jnp.where(vx1, ax * by * e, z)
    wSW = jnp.where(vx0, bx * ay * e, z)
    wSE = jnp.where(vx1, ax * ay * e, z)
    xc = jnp.clip(x0, -8, W + 7)
    P = RB * W
    wgt_ref[...] = jnp.concatenate(
        [wNW.reshape(1, P), wNE.reshape(1, P), wSW.reshape(1, P),
         wSE.reshape(1, P), jnp.zeros((4, P), jnp.float32)], axis=0)
    yx_ref[...] = jnp.concatenate(
        [y0.reshape(1, P), xc.reshape(1, P), jnp.zeros((6, P), jnp.int32)],
        axis=0)


def _prep(tenFlow, tenMetric):
    return pl.pallas_call(
        _prep_body,
        grid=(N, H // RB),
        in_specs=[pl.BlockSpec((1, 2, RB, W), lambda n, r: (n, 0, r, 0)),
                  pl.BlockSpec((1, 1, RB, W), lambda n, r: (n, 0, r, 0))],
        out_specs=[pl.BlockSpec((8, RB * W), lambda n, r: (0, n * (H // RB) + r)),
                   pl.BlockSpec((8, RB * W), lambda n, r: (0, n * (H // RB) + r))],
        out_shape=[jax.ShapeDtypeStruct((8, NHW), jnp.float32),
                   jax.ShapeDtypeStruct((8, NHW), jnp.int32)],
    )(tenFlow, tenMetric)


def _sc_splat_body(in_hbm, wgt_hbm, yx_hbm, out_hbm,
                   vbuf, wbuf, ybuf, ix0, ix1, st0, st1, zbuf, acc,
                   sem_in, sem_s0, sem_s1):
    cid = lax.axis_index("c")
    tid = lax.axis_index("s")
    zvec = jnp.zeros((16,), jnp.float32)

    @pl.loop(0, 28)
    def _zb(i):
        for t in range(CP // 16):
            zbuf[i, pl.ds(16 * t, 16)] = zvec

    @pl.loop(0, 4 * HSZ)
    def _zs(i):
        st0[i, pl.ds(CP - 16, 16)] = zvec
        st1[i, pl.ds(CP - 16, 16)] = zvec

    def start_in(pv, voff, wrow):
        pltpu.async_copy(in_hbm.at[pl.ds(pv, GSZ)],
                         vbuf.at[pl.ds(voff, GSZ)], sem_in)
        pltpu.async_copy(wgt_hbm.at[pl.ds(0, 8), pl.ds(pv, GSZ)],
                         wbuf.at[pl.ds(wrow, 8)], sem_in)
        pltpu.async_copy(yx_hbm.at[pl.ds(0, 8), pl.ds(pv, GSZ)],
                         ybuf.at[pl.ds(wrow, 8)], sem_in)

    def wait_in(pv, voff, wrow):
        pltpu.make_async_copy(in_hbm.at[pl.ds(pv, GSZ)],
                              vbuf.at[pl.ds(voff, GSZ)], sem_in).wait()
        pltpu.make_async_copy(wgt_hbm.at[pl.ds(0, 8), pl.ds(pv, GSZ)],
                              wbuf.at[pl.ds(wrow, 8)], sem_in).wait()
        pltpu.make_async_copy(yx_hbm.at[pl.ds(0, 8), pl.ds(pv, GSZ)],
                              ybuf.at[pl.ds(wrow, 8)], sem_in).wait()

    def fill_half(S, IX, h, voff, wrow, r0):
        dump = jnp.full((16,), DUMP, jnp.int32)
        e0 = jnp.where(lax.iota(jnp.int32, 16) == 0,
                       jnp.float32(1.0), jnp.float32(0.0))
        for k in range(HSZ // 16):
            cs = h * HSZ + 16 * k
            y0v = ybuf[wrow, pl.ds(cs, 16)]
            xcv = ybuf[wrow + 1, pl.ds(cs, 16)]
            la = (y0v - r0) * W + xcv + G
            okA = (y0v >= r0) & (y0v < r0 + SR)
            okB = (y0v >= r0 - 1) & (y0v < r0 + SR - 1)
            iA = jnp.where(okA, la, dump)
            iB = jnp.where(okB, la + W, dump)
            IX[0, pl.ds(16 * k, 16)] = iA
            IX[0, pl.ds(HSZ + 16 * k, 16)] = iA + 1
            IX[0, pl.ds(2 * HSZ + 16 * k, 16)] = iB
            IX[0, pl.ds(3 * HSZ + 16 * k, 16)] = iB + 1

        @pl.loop(0, HSZ // 16)
        def _grp(g):
            base = h * HSZ + g * 16
            w0v = wbuf[wrow, pl.ds(base, 16)]
            w1v = wbuf[wrow + 1, pl.ds(base, 16)]
            w2v = wbuf[wrow + 2, pl.ds(base, 16)]
            w3v = wbuf[wrow + 3, pl.ds(base, 16)]
            for k in range(16):
                qv = voff + base + k
                qs = g * 16 + k
                vvs = [vbuf[qv, pl.ds(16 * t, 16)] for t in range(C // 16)]
                w0 = w0v[k]
                w1 = w1v[k]
                w2 = w2v[k]
                w3 = w3v[k]
                for t in range(C // 16):
                    st = pl.ds(16 * t, 16)
                    S[qs, st] = vvs[t] * w0
                    S[HSZ + qs, st] = vvs[t] * w1
                    S[2 * HSZ + qs, st] = vvs[t] * w2
                    S[3 * HSZ + qs, st] = vvs[t] * w3
                S[qs, pl.ds(C, 16)] = e0 * w0
                S[HSZ + qs, pl.ds(C, 16)] = e0 * w1
                S[2 * HSZ + qs, pl.ds(C, 16)] = e0 * w2
                S[3 * HSZ + qs, pl.ds(C, 16)] = e0 * w3

    @pl.loop(0, NSLAB // 2)
    def _slab(si):
        s = si * 2 + cid
        n = s // NSLAB_IMG
        r0 = (s % NSLAB_IMG) * SR
        for zi in range(BUF_ROWS // 16 // 28):
            pltpu.async_copy(
                zbuf, acc.at[pl.ds(tid * (BUF_ROWS // 16) + zi * 28, 28)],
                sem_in)
        for zi in range(BUF_ROWS // 16 // 28):
            pltpu.make_async_copy(
                zbuf, acc.at[pl.ds(tid * (BUF_ROWS // 16) + zi * 28, 28)],
                sem_in).wait()
        plsc.subcore_barrier()
        lo = lax.clamp(0, r0 - HALO, H - SRC_ROWS)
        start = n * (H * W) + lo * W + tid * PER_TILE
        start_in(start, 0, 0)

        @pl.loop(0, NB)
        def _batch(b):
            pv = start + b * GSZ
            par = b % 2
            voff = par * GSZ
            wrow = par * 8
            wait_in(pv, voff, wrow)

            @pl.when(b + 1 < NB)
            def _():
                par2 = (b + 1) % 2
                start_in(pv + GSZ, par2 * GSZ, par2 * 8)

            @pl.loop(0, 2)
            def _hp(hp):
                for sp, (S, IX, SM) in enumerate(
                        ((st0, ix0, sem_s0), (st1, ix1, sem_s1))):
                    h = hp * 2 + sp

                    @pl.when((b > 0) | (hp > 0))
                    def _():
                        pltpu.make_async_copy(
                            S, acc.at[IX.at[0]], SM).wait()

                    fill_half(S, IX, h, voff, wrow, r0)
                    pltpu.async_copy(S, acc.at[IX.at[0]], SM, add=True)

        pltpu.make_async_copy(st0, acc.at[ix0.at[0]], sem_s0).wait()
        pltpu.make_async_copy(st1, acc.at[ix1.at[0]], sem_s1).wait()
        plsc.subcore_barrier()
        out_base = n * (H * W) + r0 * W + tid * (SR * W // 16)
        for fi in range(SR * W // 16 // GSZ):
            pltpu.async_copy(
                acc.at[pl.ds(G + tid * (SR * W // 16) + fi * GSZ, GSZ)],
                out_hbm.at[pl.ds(out_base + fi * GSZ, GSZ)], sem_in)
        for fi in range(SR * W // 16 // GSZ):
            pltpu.make_async_copy(
                acc.at[pl.ds(G + tid * (SR * W // 16) + fi * GSZ, GSZ)],
                out_hbm.at[pl.ds(out_base + fi * GSZ, GSZ)], sem_in).wait()
        plsc.subcore_barrier()


def _sc_splat(in_nhwc, wgt, yx):
    mesh = plsc.VectorSubcoreMesh(core_axis_name="c", subcore_axis_name="s")
    k = functools.partial(
        pl.kernel, mesh=mesh,
        out_type=jax.ShapeDtypeStruct((NHW, CP), jnp.float32),
        scratch_types=[
            pltpu.VMEM((2 * GSZ, C), jnp.float32),
            pltpu.VMEM((16, GSZ), jnp.float32),
            pltpu.VMEM((16, GSZ), jnp.int32),
            pltpu.VMEM((1, 4 * HSZ), jnp.int32),
            pltpu.VMEM((1, 4 * HSZ), jnp.int32),
            pltpu.VMEM((4 * HSZ, CP), jnp.float32),
            pltpu.VMEM((4 * HSZ, CP), jnp.float32),
            pltpu.VMEM((28, CP), jnp.float32),
            pltpu.VMEM_SHARED((BUF_ROWS, CP), jnp.float32),
            pltpu.SemaphoreType.DMA,
            pltpu.SemaphoreType.DMA,
            pltpu.SemaphoreType.DMA,
        ],    )(_sc_splat_body)
    return k(in_nhwc, wgt, yx)


def _post_body(acc_ref, out_ref):
    a = acc_ref[...]
    norm = a[:, C:C + 1] + 1e-7
    out_ref[...] = a[:, :C] / norm


def _post(outp):
    RB2 = 1024
    return pl.pallas_call(
        _post_body,
        grid=(NHW // RB2,),
        in_specs=[pl.BlockSpec((RB2, CP), lambda i: (i, 0))],
        out_specs=pl.BlockSpec((RB2, C), lambda i: (i, 0)),
        out_shape=jax.ShapeDtypeStruct((NHW, C), jnp.float32),
    )(outp)


def kernel(tenIn, tenFlow, tenMetric):
    in_nhwc = tenIn.transpose(0, 2, 3, 1).reshape(NHW, C)
    wgt, yx = _prep(tenFlow, tenMetric)
    outp = _sc_splat(in_nhwc, wgt, yx)
    res = _post(outp)
    return res.reshape(N, H, W, C).transpose(0, 3, 1, 2)


# speedup vs baseline: 3.9595x; 1.1331x over previous
import functools

import jax
import jax.numpy as jnp
from jax import lax
from jax.experimental import pallas as pl
from jax.experimental.pallas import tpu as pltpu
from jax.experimental.pallas import tpu_sc as plsc

N, C, H, W = 2, 96, 384, 384
CP = 128
NHW = N * H * W
SR = 16
G = 16
HALO = 8
NSLAB_IMG = H // SR
NSLAB = N * NSLAB_IMG
DUMP = G + SR * W + 32
BUF_ROWS = 6272
GSZ = 128
HSZ = 32
SRC_ROWS = SR + 2 * HALO
PER_TILE = SRC_ROWS * W // 16
NB = PER_TILE // GSZ
RB = 8


def _prep_body(in_ref, flow_ref, met_ref, nhwc_ref, wgt_ref, yx_ref):
    rc = pl.program_id(1)
    fx = flow_ref[0, 0]
    fy = flow_ref[0, 1]
    m = met_ref[0, 0]
    xg = lax.broadcasted_iota(jnp.int32, (RB, W), 1).astype(jnp.float32)
    yg = (lax.broadcasted_iota(jnp.int32, (RB, W), 0)
          .astype(jnp.float32) + (rc * RB).astype(jnp.float32))
    fltx = xg + fx
    flty = yg + fy
    x0f = jnp.floor(fltx)
    y0f = jnp.floor(flty)
    x0 = x0f.astype(jnp.int32)
    y0 = y0f.astype(jnp.int32)
    ax = fltx - x0f
    ay = flty - y0f
    e = jnp.exp(m)
    vx0 = (x0 >= 0) & (x0 < W)
    vx1 = (x0 >= -1) & (x0 < W - 1)
    bx = 1.0 - ax
    by = 1.0 - ay
    z = jnp.zeros_like(ax)
    wNW = jnp.where(vx0, bx * by * e, z)
    wNE = jnp.where(vx1, ax * by * e, z)
    wSW = jnp.where(vx0, bx * ay * e, z)
    wSE = jnp.where(vx1, ax * ay * e, z)
    xc = jnp.clip(x0, -8, W + 7)
    P = RB * W
    nhwc_ref[...] = jnp.transpose(in_ref[0].reshape(C, P), (1, 0))
    wgt_ref[...] = jnp.concatenate(
        [wNW.reshape(1, P), wNE.reshape(1, P), wSW.reshape(1, P),
         wSE.reshape(1, P), jnp.zeros((4, P), jnp.float32)], axis=0)
    yx_ref[...] = jnp.concatenate(
        [y0.reshape(1, P), xc.reshape(1, P), jnp.zeros((6, P), jnp.int32)],
        axis=0)


def _prep(tenIn, tenFlow, tenMetric):
    return pl.pallas_call(
        _prep_body,
        grid=(N, H // RB),
        in_specs=[pl.BlockSpec((1, C, RB, W), lambda n, r: (n, 0, r, 0)),
                  pl.BlockSpec((1, 2, RB, W), lambda n, r: (n, 0, r, 0)),
                  pl.BlockSpec((1, 1, RB, W), lambda n, r: (n, 0, r, 0))],
        out_specs=[pl.BlockSpec((RB * W, C), lambda n, r: (n * (H // RB) + r, 0)),
                   pl.BlockSpec((8, RB * W), lambda n, r: (0, n * (H // RB) + r)),
                   pl.BlockSpec((8, RB * W), lambda n, r: (0, n * (H // RB) + r))],
        out_shape=[jax.ShapeDtypeStruct((NHW, C), jnp.float32),
                   jax.ShapeDtypeStruct((8, NHW), jnp.float32),
                   jax.ShapeDtypeStruct((8, NHW), jnp.int32)],
    )(tenIn, tenFlow, tenMetric)


def _sc_splat_body(in_hbm, wgt_hbm, yx_hbm, out_hbm,
                   vbuf, wbuf, ybuf, ix0, ix1, st0, st1, zbuf, acc,
                   sem_in, sem_s0, sem_s1):
    cid = lax.axis_index("c")
    tid = lax.axis_index("s")
    zvec = jnp.zeros((16,), jnp.float32)

    @pl.loop(0, 28)
    def _zb(i):
        for t in range(CP // 16):
            zbuf[i, pl.ds(16 * t, 16)] = zvec

    @pl.loop(0, 4 * HSZ)
    def _zs(i):
        st0[i, pl.ds(CP - 16, 16)] = zvec
        st1[i, pl.ds(CP - 16, 16)] = zvec

    def start_in(pv, voff, wrow):
        pltpu.async_copy(in_hbm.at[pl.ds(pv, GSZ)],
                         vbuf.at[pl.ds(voff, GSZ)], sem_in)
        pltpu.async_copy(wgt_hbm.at[pl.ds(0, 8), pl.ds(pv, GSZ)],
                         wbuf.at[pl.ds(wrow, 8)], sem_in)
        pltpu.async_copy(yx_hbm.at[pl.ds(0, 8), pl.ds(pv, GSZ)],
                         ybuf.at[pl.ds(wrow, 8)], sem_in)

    def wait_in(pv, voff, wrow):
        pltpu.make_async_copy(in_hbm.at[pl.ds(pv, GSZ)],
                              vbuf.at[pl.ds(voff, GSZ)], sem_in).wait()
        pltpu.make_async_copy(wgt_hbm.at[pl.ds(0, 8), pl.ds(pv, GSZ)],
                              wbuf.at[pl.ds(wrow, 8)], sem_in).wait()
        pltpu.make_async_copy(yx_hbm.at[pl.ds(0, 8), pl.ds(pv, GSZ)],
                              ybuf.at[pl.ds(wrow, 8)], sem_in).wait()

    def fill_half(S, IX, h, voff, wrow, r0):
        dump = jnp.full((16,), DUMP, jnp.int32)
        e0 = jnp.where(lax.iota(jnp.int32, 16) == 0,
                       jnp.float32(1.0), jnp.float32(0.0))
        for k in range(HSZ // 16):
            cs = h * HSZ + 16 * k
            y0v = ybuf[wrow, pl.ds(cs, 16)]
            xcv = ybuf[wrow + 1, pl.ds(cs, 16)]
            la = (y0v - r0) * W + xcv + G
            okA = (y0v >= r0) & (y0v < r0 + SR)
            okB = (y0v >= r0 - 1) & (y0v < r0 + SR - 1)
            iA = jnp.where(okA, la, dump)
            iB = jnp.where(okB, la + W, dump)
            IX[0, pl.ds(16 * k, 16)] = iA
            IX[0, pl.ds(HSZ + 16 * k, 16)] = iA + 1
            IX[0, pl.ds(2 * HSZ + 16 * k, 16)] = iB
            IX[0, pl.ds(3 * HSZ + 16 * k, 16)] = iB + 1

        @pl.loop(0, HSZ // 16)
        def _grp(g):
            base = h * HSZ + g * 16
            w0v = wbuf[wrow, pl.ds(base, 16)]
            w1v = wbuf[wrow + 1, pl.ds(base, 16)]
            w2v = wbuf[wrow + 2, pl.ds(base, 16)]
            w3v = wbuf[wrow + 3, pl.ds(base, 16)]
            for k in range(16):
                qv = voff + base + k
                qs = g * 16 + k
                vvs = [vbuf[qv, pl.ds(16 * t, 16)] for t in range(C // 16)]
                w0 = w0v[k]
                w1 = w1v[k]
                w2 = w2v[k]
                w3 = w3v[k]
                for t in range(C // 16):
                    st = pl.ds(16 * t, 16)
                    S[qs, st] = vvs[t] * w0
                    S[HSZ + qs, st] = vvs[t] * w1
                    S[2 * HSZ + qs, st] = vvs[t] * w2
                    S[3 * HSZ + qs, st] = vvs[t] * w3
                S[qs, pl.ds(C, 16)] = e0 * w0
                S[HSZ + qs, pl.ds(C, 16)] = e0 * w1
                S[2 * HSZ + qs, pl.ds(C, 16)] = e0 * w2
                S[3 * HSZ + qs, pl.ds(C, 16)] = e0 * w3

    @pl.loop(0, NSLAB // 2)
    def _slab(si):
        s = si * 2 + cid
        n = s // NSLAB_IMG
        r0 = (s % NSLAB_IMG) * SR
        for zi in range(BUF_ROWS // 16 // 28):
            pltpu.async_copy(
                zbuf, acc.at[pl.ds(tid * (BUF_ROWS // 16) + zi * 28, 28)],
                sem_in)
        for zi in range(BUF_ROWS // 16 // 28):
            pltpu.make_async_copy(
                zbuf, acc.at[pl.ds(tid * (BUF_ROWS // 16) + zi * 28, 28)],
                sem_in).wait()
        plsc.subcore_barrier()
        lo = lax.clamp(0, r0 - HALO, H - SRC_ROWS)
        start = n * (H * W) + lo * W + tid * PER_TILE
        start_in(start, 0, 0)

        @pl.loop(0, NB)
        def _batch(b):
            pv = start + b * GSZ
            par = b % 2
            voff = par * GSZ
            wrow = par * 8
            wait_in(pv, voff, wrow)

            @pl.when(b + 1 < NB)
            def _():
                par2 = (b + 1) % 2
                start_in(pv + GSZ, par2 * GSZ, par2 * 8)

            @pl.loop(0, 2)
            def _hp(hp):
                for sp, (S, IX, SM) in enumerate(
                        ((st0, ix0, sem_s0), (st1, ix1, sem_s1))):
                    h = hp * 2 + sp

                    @pl.when((b > 0) | (hp > 0))
                    def _():
                        pltpu.make_async_copy(
                            S, acc.at[IX.at[0]], SM).wait()

                    fill_half(S, IX, h, voff, wrow, r0)
                    pltpu.async_copy(S, acc.at[IX.at[0]], SM, add=True)

        pltpu.make_async_copy(st0, acc.at[ix0.at[0]], sem_s0).wait()
        pltpu.make_async_copy(st1, acc.at[ix1.at[0]], sem_s1).wait()
        plsc.subcore_barrier()
        out_base = n * (H * W) + r0 * W + tid * (SR * W // 16)
        for fi in range(SR * W // 16 // GSZ):
            pltpu.async_copy(
                acc.at[pl.ds(G + tid * (SR * W // 16) + fi * GSZ, GSZ)],
                out_hbm.at[pl.ds(out_base + fi * GSZ, GSZ)], sem_in)
        for fi in range(SR * W // 16 // GSZ):
            pltpu.make_async_copy(
                acc.at[pl.ds(G + tid * (SR * W // 16) + fi * GSZ, GSZ)],
                out_hbm.at[pl.ds(out_base + fi * GSZ, GSZ)], sem_in).wait()
        plsc.subcore_barrier()


def _sc_splat(in_nhwc, wgt, yx):
    mesh = plsc.VectorSubcoreMesh(core_axis_name="c", subcore_axis_name="s")
    k = functools.partial(
        pl.kernel, mesh=mesh,
        out_type=jax.ShapeDtypeStruct((NHW, CP), jnp.float32),
        scratch_types=[
            pltpu.VMEM((2 * GSZ, C), jnp.float32),
            pltpu.VMEM((16, GSZ), jnp.float32),
            pltpu.VMEM((16, GSZ), jnp.int32),
            pltpu.VMEM((1, 4 * HSZ), jnp.int32),
            pltpu.VMEM((1, 4 * HSZ), jnp.int32),
            pltpu.VMEM((4 * HSZ, CP), jnp.float32),
            pltpu.VMEM((4 * HSZ, CP), jnp.float32),
            pltpu.VMEM((28, CP), jnp.float32),
            pltpu.VMEM_SHARED((BUF_ROWS, CP), jnp.float32),
            pltpu.SemaphoreType.DMA,
            pltpu.SemaphoreType.DMA,
            pltpu.SemaphoreType.DMA,
        ],    )(_sc_splat_body)
    return k(in_nhwc, wgt, yx)


def _post_body(acc_ref, out_ref):
    a = acc_ref[...]
    norm = a[:, C:C + 1] + 1e-7
    res = a[:, :C] / norm
    out_ref[...] = jnp.transpose(res, (1, 0)).reshape(1, C, RB, W)


def _post(outp):
    return pl.pallas_call(
        _post_body,
        grid=(N, H // RB),
        in_specs=[pl.BlockSpec((RB * W, CP), lambda n, r: (n * (H // RB) + r, 0))],
        out_specs=pl.BlockSpec((1, C, RB, W), lambda n, r: (n, 0, r, 0)),
        out_shape=jax.ShapeDtypeStruct((N, C, H, W), jnp.float32),
    )(outp)


def kernel(tenIn, tenFlow, tenMetric):
    in_nhwc, wgt, yx = _prep(tenIn, tenFlow, tenMetric)
    outp = _sc_splat(in_nhwc, wgt, yx)
    return _post(outp)
